# Initial kernel scaffold; baseline (speedup 1.0000x reference)
import functools

import jax
import jax.numpy as jnp
from jax import lax
from jax.experimental import pallas as pl
from jax.experimental.pallas import tpu as pltpu
from jax.experimental.pallas import tpu_sc as plsc

N = 50000
D = 64
E = 800000
B = 4096
K = 40
REG = 0.0001

NHALF = 25088
NPAD = 2 * NHALF
ACC = 25600
EPAD = 819200
W = 128
PER_TILE_E = EPAD // 16
NCHUNK = PER_TILE_E // W

_mesh = lambda: plsc.VectorSubcoreMesh(core_axis_name="c", subcore_axis_name="s")


def _local_idx(colbuf, idxbuf, base):
    lanes = lax.iota(jnp.int32, 16)
    for k in range(W // 16):
        sl = pl.ds(k * 16, 16)
        cv = colbuf[sl]
        loc = cv - base
        m = (loc >= 0) & (loc < NHALF)
        idxbuf[sl] = jnp.where(m, loc, NHALF + lanes)


def _sc_degree(colp):
    @functools.partial(
        pl.kernel,
        out_type=jax.ShapeDtypeStruct((NPAD,), jnp.float32),
        mesh=_mesh(),
        scratch_types=[
            pltpu.VMEM((W,), jnp.int32),
            pltpu.VMEM((W,), jnp.int32),
            pltpu.VMEM((W,), jnp.float32),
            pltpu.VMEM((1600,), jnp.float32),
            pltpu.VMEM_SHARED((ACC,), jnp.float32),
        ],
    )
    def k(col_hbm, deg_hbm, colbuf, idxbuf, ones, zbuf, acc):
        c = lax.axis_index("c")
        s = lax.axis_index("s")
        base = c * NHALF
        for k16 in range(1600 // 16):
            zbuf[pl.ds(k16 * 16, 16)] = jnp.zeros((16,), jnp.float32)
        for k16 in range(W // 16):
            ones[pl.ds(k16 * 16, 16)] = jnp.ones((16,), jnp.float32)
        pltpu.sync_copy(zbuf, acc.at[pl.ds(s * 1600, 1600)])
        plsc.subcore_barrier()

        @pl.loop(0, NCHUNK)
        def _(g):
            e0 = s * PER_TILE_E + g * W
            pltpu.sync_copy(col_hbm.at[pl.ds(e0, W)], colbuf)
            _local_idx(colbuf, idxbuf, base)
            pltpu.sync_copy(ones, acc.at[idxbuf], add=True)

        plsc.subcore_barrier()
        pltpu.sync_copy(
            acc.at[pl.ds(s * 1568, 1568)],
            deg_hbm.at[pl.ds(base + s * 1568, 1568)],
        )

    return k(colp)


def _sc_spass(a, colp, rowp):
    @functools.partial(
        pl.kernel,
        out_type=jax.ShapeDtypeStruct((NPAD, D), jnp.float32),
        mesh=_mesh(),
        scratch_types=[
            pltpu.VMEM((W,), jnp.int32),
            pltpu.VMEM((W,), jnp.int32),
            pltpu.VMEM((W,), jnp.int32),
            pltpu.VMEM((W, D), jnp.float32),
            pltpu.VMEM((100, D), jnp.float32),
            pltpu.VMEM_SHARED((ACC, D), jnp.float32),
        ],
    )
    def k(a_hbm, col_hbm, row_hbm, y_hbm, colbuf, rowbuf, idxbuf, gbuf, zbuf, acc):
        c = lax.axis_index("c")
        s = lax.axis_index("s")
        base = c * NHALF
        for r in range(100):
            for k16 in range(D // 16):
                zbuf[r, pl.ds(k16 * 16, 16)] = jnp.zeros((16,), jnp.float32)
        for j in range(16):
            pltpu.sync_copy(zbuf, acc.at[pl.ds(s * 1600 + j * 100, 100)])
        plsc.subcore_barrier()

        @pl.loop(0, NCHUNK)
        def _(g):
            e0 = s * PER_TILE_E + g * W
            pltpu.sync_copy(col_hbm.at[pl.ds(e0, W)], colbuf)
            pltpu.sync_copy(row_hbm.at[pl.ds(e0, W)], rowbuf)
            _local_idx(colbuf, idxbuf, base)
            pltpu.sync_copy(a_hbm.at[rowbuf], gbuf)
            pltpu.sync_copy(gbuf, acc.at[idxbuf], add=True)

        plsc.subcore_barrier()
        pltpu.sync_copy(
            acc.at[pl.ds(s * 1568, 1568)],
            y_hbm.at[pl.ds(base + s * 1568, 1568)],
        )

    return k(a, colp, rowp)


def _sc_gather(Z, idx_all):
    NUV = 2 * B
    NNEG = B * K
    TOT = NUV + NNEG
    WIN_PER_TILE = TOT // (32 * W)
    UV_WINS = NUV // W

    @functools.partial(
        pl.kernel,
        out_type=(
            jax.ShapeDtypeStruct((NUV, D), jnp.float32),
            jax.ShapeDtypeStruct((NNEG, D), jnp.float32),
        ),
        mesh=_mesh(),
        scratch_types=[
            pltpu.VMEM((W,), jnp.int32),
            pltpu.VMEM((W, D), jnp.float32),
        ],
    )
    def k(z_hbm, idx_hbm, uv_hbm, n_hbm, idxbuf, gbuf):
        c = lax.axis_index("c")
        s = lax.axis_index("s")
        wid = s * 2 + c

        @pl.loop(0, WIN_PER_TILE)
        def _(j):
            wabs = wid * WIN_PER_TILE + j
            pltpu.sync_copy(idx_hbm.at[pl.ds(wabs * W, W)], idxbuf)
            pltpu.sync_copy(z_hbm.at[idxbuf], gbuf)

            @pl.when(wabs < UV_WINS)
            def _():
                pltpu.sync_copy(gbuf, uv_hbm.at[pl.ds(wabs * W, W)])

            @pl.when(wabs >= UV_WINS)
            def _():
                pltpu.sync_copy(gbuf, n_hbm.at[pl.ds(wabs * W - NUV, W)])

    return k(Z, idx_all)


def _tc_scale0(deg2d, E1p):
    def body(deg_ref, e1_ref, dis_ref, dis2_ref, a1_ref):
        d = deg_ref[...]
        di = jnp.where(d > 0, 1.0 / jnp.sqrt(jnp.maximum(d, 1.0)), 0.0)
        dis_ref[...] = di
        dis2_ref[...] = di * di
        a1_ref[...] = di * e1_ref[...]

    R = 256
    return pl.pallas_call(
        body,
        grid=(NPAD // R,),
        in_specs=[
            pl.BlockSpec((R, 1), lambda i: (i, 0)),
            pl.BlockSpec((R, D), lambda i: (i, 0)),
        ],
        out_specs=[
            pl.BlockSpec((R, 1), lambda i: (i, 0)),
            pl.BlockSpec((R, 1), lambda i: (i, 0)),
            pl.BlockSpec((R, D), lambda i: (i, 0)),
        ],
        out_shape=[
            jax.ShapeDtypeStruct((NPAD, 1), jnp.float32),
            jax.ShapeDtypeStruct((NPAD, 1), jnp.float32),
            jax.ShapeDtypeStruct((NPAD, D), jnp.float32),
        ],
    )(deg2d, E1p)


def _tc_scale1(dis2, y1):
    def body(dis2_ref, y_ref, a_ref):
        a_ref[...] = dis2_ref[...] * y_ref[...]

    R = 256
    return pl.pallas_call(
        body,
        grid=(NPAD // R,),
        in_specs=[
            pl.BlockSpec((R, 1), lambda i: (i, 0)),
            pl.BlockSpec((R, D), lambda i: (i, 0)),
        ],
        out_specs=pl.BlockSpec((R, D), lambda i: (i, 0)),
        out_shape=jax.ShapeDtypeStruct((NPAD, D), jnp.float32),
    )(dis2, y1)


def _tc_fuse(E1p, y1, y2, dis, E2p, W0, b0, W1, b1, attn_W, attn_b, q_W):
    def body(e1_ref, y1_ref, y2_ref, dis_ref, e2_ref, w0_ref, b0_ref, w1_ref,
             b1_ref, aw_ref, ab_ref, qw_ref, z_ref):
        di = dis_ref[...]
        z_p = (e1_ref[...] + di * y1_ref[...] + di * y2_ref[...]) / 3.0
        h = jnp.maximum(
            jnp.dot(e2_ref[...], w0_ref[...],
                    preferred_element_type=jnp.float32) + b0_ref[...], 0.0)
        z_n = jnp.maximum(
            jnp.dot(h, w1_ref[...],
                    preferred_element_type=jnp.float32) + b1_ref[...], 0.0)
        aw = aw_ref[...]
        ab = ab_ref[...]
        qw = qw_ref[...]
        w_p = jnp.dot(jnp.tanh(
            jnp.dot(z_p, aw, preferred_element_type=jnp.float32) + ab), qw,
            preferred_element_type=jnp.float32)
        w_n = jnp.dot(jnp.tanh(
            jnp.dot(z_n, aw, preferred_element_type=jnp.float32) + ab), qw,
            preferred_element_type=jnp.float32)
        m = jnp.maximum(w_p, w_n)
        e_p = jnp.exp(w_p - m)
        e_n = jnp.exp(w_n - m)
        alpha = e_p / (e_p + e_n)
        z_ref[...] = alpha * z_p + (1.0 - alpha) * z_n

    R = 512
    full = lambda shape: pl.BlockSpec(shape, lambda i: (0, 0))
    return pl.pallas_call(
        body,
        grid=(NPAD // R,),
        in_specs=[
            pl.BlockSpec((R, D), lambda i: (i, 0)),
            pl.BlockSpec((R, D), lambda i: (i, 0)),
            pl.BlockSpec((R, D), lambda i: (i, 0)),
            pl.BlockSpec((R, 1), lambda i: (i, 0)),
            pl.BlockSpec((R, D), lambda i: (i, 0)),
            full((D, D)), full((1, D)), full((D, D)), full((1, D)),
            full((D, D)), full((1, D)), full((D, 1)),
        ],
        out_specs=pl.BlockSpec((R, D), lambda i: (i, 0)),
        out_shape=jax.ShapeDtypeStruct((NPAD, D), jnp.float32),
    )(E1p, y1, y2, dis, E2p, W0, b0, W1, b1, attn_W, attn_b, q_W)


def _tc_loss(uv, nmat, w2d):
    R = 256

    def body(u_ref, v_ref, n_ref, w_ref, out_ref):
        i = pl.program_id(0)
        u = u_ref[...]
        v = v_ref[...]
        nb = n_ref[...].reshape(R, K, D)
        wv = w_ref[...]
        pos = jnp.sum(u * v, axis=1, keepdims=True)
        neg = jnp.sum(u[:, None, :] * nb, axis=2)
        coef = -0.5 * jnp.sign(wv) + 1.5
        x = coef * pos - neg
        bpr = jax.nn.log_sigmoid(x)
        reg = jnp.sum(u * u) + jnp.sum(v * v) + jnp.sum(n_ref[...] * n_ref[...])
        part = -jnp.sum(bpr) + REG * reg

        @pl.when(i == 0)
        def _():
            out_ref[0, 0] = 0.0

        out_ref[0, 0] += part

    return pl.pallas_call(
        body,
        grid=(B // R,),
        in_specs=[
            pl.BlockSpec((R, D), lambda i: (i, 0)),
            pl.BlockSpec((R, D), lambda i: (16 + i, 0)),
            pl.BlockSpec((R * K, D), lambda i: (i, 0)),
            pl.BlockSpec((R, 1), lambda i: (i, 0)),
        ],
        out_specs=pl.BlockSpec((1, 1), lambda i: (0, 0)),
        out_shape=jax.ShapeDtypeStruct((1, 1), jnp.float32),
    )(uv, uv, nmat, w2d)


def kernel(E1, E2, W0, b0, W1, b1, attn_W, attn_b, q_W, w, edge_index, u, v, n):
    row = edge_index[0].astype(jnp.int32)
    col = edge_index[1].astype(jnp.int32)
    npad_e = EPAD - E
    pad_row = (jnp.arange(npad_e, dtype=jnp.int32) % 16) + N
    pad_col = jnp.full((npad_e,), -1, jnp.int32)
    rowp = jnp.concatenate([row, pad_row])
    colp = jnp.concatenate([col, pad_col])

    E1p = jnp.pad(E1, ((0, NPAD - N), (0, 0)))
    E2p = jnp.pad(E2, ((0, NPAD - N), (0, 0)))

    deg = _sc_degree(colp)
    dis, dis2, a1 = _tc_scale0(deg.reshape(NPAD, 1), E1p)
    y1 = _sc_spass(a1, colp, rowp)
    a2 = _tc_scale1(dis2, y1)
    y2 = _sc_spass(a2, colp, rowp)
    Z = _tc_fuse(E1p, y1, y2, dis, E2p,
                 W0, b0.reshape(1, D), W1, b1.reshape(1, D),
                 attn_W, attn_b.reshape(1, D), q_W)

    idx_all = jnp.concatenate([
        u.astype(jnp.int32), v.astype(jnp.int32),
        n.reshape(-1).astype(jnp.int32)])
    uv, nmat = _sc_gather(Z, idx_all)

    out = _tc_loss(uv, nmat, w.reshape(B, 1))
    return out[0, 0]


# speedup vs baseline: 16.9677x; 16.9677x over previous
import functools

import jax
import jax.numpy as jnp
from jax import lax
from jax.experimental import pallas as pl
from jax.experimental.pallas import tpu as pltpu
from jax.experimental.pallas import tpu_sc as plsc

N = 50000
D = 64
E = 800000
B = 4096
K = 40
REG = 0.0001

NPAD = 50176
HD = D // 2
ACC = 50688
EPAD = 819200
W = 128
NCHUNK = EPAD // 16 // W
DCHUNK = EPAD // 32 // W

_mesh = lambda: plsc.VectorSubcoreMesh(core_axis_name="c", subcore_axis_name="s")
_SC_PARAMS = pltpu.CompilerParams(use_tc_tiling_on_sc=False)


def _local_idx(colbuf, idxbuf):
    lanes = lax.iota(jnp.int32, 16)
    for k in range(W // 16):
        sl = pl.ds(k * 16, 16)
        cv = colbuf[sl]
        idxbuf[sl] = jnp.where(cv >= 0, cv, NPAD + lanes)


def _wait(src, dst, sem):
    pltpu.make_async_copy(src, dst, sem).wait()


def _sc_degree(colp):
    @functools.partial(
        pl.kernel,
        out_type=jax.ShapeDtypeStruct((2 * NPAD,), jnp.float32),
        mesh=_mesh(),
        compiler_params=_SC_PARAMS,
        scratch_types=[
            pltpu.VMEM((2, W), jnp.int32),
            pltpu.VMEM((2, W), jnp.int32),
            pltpu.VMEM((W,), jnp.float32),
            pltpu.VMEM((3168,), jnp.float32),
            pltpu.VMEM_SHARED((ACC,), jnp.float32),
            pltpu.SemaphoreType.DMA,
            pltpu.SemaphoreType.DMA,
            pltpu.SemaphoreType.DMA,
            pltpu.SemaphoreType.DMA,
        ],
    )
    def k(col_hbm, deg_hbm, colbuf, idxbuf, ones, zbuf, acc,
          lsem0, lsem1, ssem0, ssem1):
        c = lax.axis_index("c")
        s = lax.axis_index("s")

        @pl.loop(0, 3168 // 16)
        def _(k16):
            zbuf[pl.ds(k16 * 16, 16)] = jnp.zeros((16,), jnp.float32)

        for k16 in range(W // 16):
            ones[pl.ds(k16 * 16, 16)] = jnp.ones((16,), jnp.float32)
        pltpu.sync_copy(zbuf, acc.at[pl.ds(s * 3168, 3168)])
        plsc.subcore_barrier()

        def load(b, g, ls):
            e0 = c * (EPAD // 2) + s * (DCHUNK * W) + g * W
            pltpu.async_copy(col_hbm.at[pl.ds(e0, W)], colbuf.at[b], ls)

        load(0, 0, lsem0)
        load(1, 1, lsem1)

        @pl.loop(0, DCHUNK // 2)
        def _(t):
            for b, ls, ss in ((0, lsem0, ssem0), (1, lsem1, ssem1)):
                g = t * 2 + b
                _wait(col_hbm.at[pl.ds(0, W)], colbuf.at[b], ls)

                @pl.when(t > 0)
                def _():
                    _wait(ones, acc.at[idxbuf.at[b]], ss)

                _local_idx(colbuf.at[b], idxbuf.at[b])
                pltpu.async_copy(ones, acc.at[idxbuf.at[b]], ss, add=True)

                @pl.when(t < DCHUNK // 2 - 1)
                def _():
                    load(b, g + 2, ls)

        _wait(ones, acc.at[idxbuf.at[0]], ssem0)
        _wait(ones, acc.at[idxbuf.at[1]], ssem1)
        plsc.subcore_barrier()
        pltpu.sync_copy(acc.at[pl.ds(s * 3136, 3136)], zbuf.at[pl.ds(0, 3136)])
        pltpu.sync_copy(zbuf.at[pl.ds(0, 3136)],
                        deg_hbm.at[pl.ds(c * NPAD + s * 3136, 3136)])

    return k(colp)


def _sc_spass(a2, colp, rowp):
    @functools.partial(
        pl.kernel,
        out_type=jax.ShapeDtypeStruct((2 * NPAD, HD), jnp.float32),
        mesh=_mesh(),
        compiler_params=_SC_PARAMS,
        scratch_types=[
            pltpu.VMEM((4, W), jnp.int32),
            pltpu.VMEM((4, W), jnp.int32),
            pltpu.VMEM((2, W), jnp.int32),
            pltpu.VMEM((2, W, HD), jnp.float32),
            pltpu.VMEM((96, HD), jnp.float32),
            pltpu.VMEM_SHARED((ACC, HD), jnp.float32),
            pltpu.SemaphoreType.DMA,
            pltpu.SemaphoreType.DMA,
            pltpu.SemaphoreType.DMA,
            pltpu.SemaphoreType.DMA,
            pltpu.SemaphoreType.DMA,
            pltpu.SemaphoreType.DMA,
            pltpu.SemaphoreType.DMA,
            pltpu.SemaphoreType.DMA,
        ],
    )
    def k(a_hbm, col_hbm, row_hbm, y_hbm, colbuf, rowbuf, idxbuf, gbuf, zbuf,
          acc, lsem0, lsem1, lsem2, lsem3, gsem0, gsem1, ssem0, ssem1):
        lsem = (lsem0, lsem1, lsem2, lsem3)
        gsem = (gsem0, gsem1)
        ssem = (ssem0, ssem1)
        c = lax.axis_index("c")
        s = lax.axis_index("s")
        roff = c * NPAD

        @pl.loop(0, 96)
        def _(r):
            for k16 in range(HD // 16):
                zbuf[r, pl.ds(k16 * 16, 16)] = jnp.zeros((16,), jnp.float32)

        @pl.loop(0, 3168 // 96)
        def _(j):
            pltpu.sync_copy(zbuf, acc.at[pl.ds(s * 3168 + j * 96, 96)])

        plsc.subcore_barrier()

        def load(slot, g):
            e0 = s * (NCHUNK * W) + g * W
            pltpu.async_copy(col_hbm.at[pl.ds(e0, W)], colbuf.at[slot],
                             lsem[slot])
            pltpu.async_copy(row_hbm.at[pl.ds(e0, W)], rowbuf.at[slot],
                             lsem[slot])

        def wait_load(slot):
            _wait(col_hbm.at[pl.ds(0, W)], colbuf.at[slot], lsem[slot])
            _wait(row_hbm.at[pl.ds(0, W)], rowbuf.at[slot], lsem[slot])

        def wait_scat(b2):
            _wait(gbuf.at[b2], acc.at[idxbuf.at[b2]], ssem[b2])

        def issue_scat(b2):
            pltpu.async_copy(gbuf.at[b2], acc.at[idxbuf.at[b2]], ssem[b2],
                             add=True)

        def wait_gather(slot, b2):
            _wait(a_hbm.at[rowbuf.at[slot]], gbuf.at[b2], gsem[b2])

        def adjust_rows(slot):
            for k16 in range(W // 16):
                sl = pl.ds(k16 * 16, 16)
                rowbuf[slot, sl] = rowbuf[slot, sl] + roff

        load(0, 0)
        load(1, 1)
        load(2, 2)

        @pl.loop(0, NCHUNK // 4)
        def _(t):
            for u in range(4):
                g = t * 4 + u
                b2 = u % 2
                pu = (u + 3) % 4
                wait_load(u)
                if u >= 2:
                    wait_scat(b2)
                else:
                    @pl.when(t > 0)
                    def _():
                        wait_scat(b2)
                _local_idx(colbuf.at[u], idxbuf.at[b2])
                adjust_rows(u)
                pltpu.async_copy(a_hbm.at[rowbuf.at[u]], gbuf.at[b2],
                                 gsem[b2])
                if u >= 1:
                    wait_gather(pu, 1 - b2)
                    issue_scat(1 - b2)
                    @pl.when(g + 3 < NCHUNK)
                    def _():
                        load(pu, g + 3)
                else:
                    @pl.when(t > 0)
                    def _():
                        wait_gather(3, 1)
                        issue_scat(1)
                    load(3, g + 3)

        wait_gather(3, 1)
        issue_scat(1)
        wait_scat(0)
        wait_scat(1)
        plsc.subcore_barrier()
        pltpu.sync_copy(
            acc.at[pl.ds(s * 3136, 3136)],
            y_hbm.at[pl.ds(c * NPAD + s * 3136, 3136)],
        )

    return k(a2, colp, rowp)


def _sc_gather(Z, idx_all):
    NUV = 2 * B
    NNEG = B * K
    TOT = NUV + NNEG
    WIN_PER_TILE = TOT // (32 * W)
    UV_WINS = NUV // W

    @functools.partial(
        pl.kernel,
        out_type=(
            jax.ShapeDtypeStruct((NUV, D), jnp.float32),
            jax.ShapeDtypeStruct((NNEG, D), jnp.float32),
        ),
        mesh=_mesh(),
        compiler_params=_SC_PARAMS,
        scratch_types=[
            pltpu.VMEM((2, W), jnp.int32),
            pltpu.VMEM((2, W, D), jnp.float32),
            pltpu.SemaphoreType.DMA,
            pltpu.SemaphoreType.DMA,
            pltpu.SemaphoreType.DMA,
            pltpu.SemaphoreType.DMA,
            pltpu.SemaphoreType.DMA,
            pltpu.SemaphoreType.DMA,
        ],
    )
    def k(z_hbm, idx_hbm, uv_hbm, n_hbm, idxbuf, gbuf,
          lsem0, lsem1, gsem0, gsem1, osem0, osem1):
        c = lax.axis_index("c")
        s = lax.axis_index("s")
        wid = s * 2 + c

        def load(b, j, ls):
            wabs = wid * WIN_PER_TILE + j
            pltpu.async_copy(idx_hbm.at[pl.ds(wabs * W, W)], idxbuf.at[b], ls)

        load(0, 0, lsem0)
        load(1, 1, lsem1)

        @pl.loop(0, WIN_PER_TILE // 2)
        def _(t):
            for b, ls, gs, os in ((0, lsem0, gsem0, osem0),
                                  (1, lsem1, gsem1, osem1)):
                j = t * 2 + b
                wabs = wid * WIN_PER_TILE + j
                _wait(idx_hbm.at[pl.ds(0, W)], idxbuf.at[b], ls)

                @pl.when(t > 0)
                def _():
                    _wait(gbuf.at[b], uv_hbm.at[pl.ds(0, W)], os)

                pltpu.async_copy(z_hbm.at[idxbuf.at[b]], gbuf.at[b], gs)
                _wait(z_hbm.at[idxbuf.at[b]], gbuf.at[b], gs)

                @pl.when(wabs < UV_WINS)
                def _():
                    pltpu.async_copy(gbuf.at[b], uv_hbm.at[pl.ds(wabs * W, W)],
                                     os)

                @pl.when(wabs >= UV_WINS)
                def _():
                    pltpu.async_copy(
                        gbuf.at[b], n_hbm.at[pl.ds(wabs * W - NUV, W)], os)

                @pl.when(t < WIN_PER_TILE // 2 - 1)
                def _():
                    load(b, j + 2, ls)

        _wait(gbuf.at[0], uv_hbm.at[pl.ds(0, W)], osem0)
        _wait(gbuf.at[1], uv_hbm.at[pl.ds(0, W)], osem1)

    return k(Z, idx_all)


def _tc_scale0(deg2d, E1c):
    def body(degA_ref, degB_ref, e1_ref, dis_ref, dis2_ref, a1_ref):
        d = degA_ref[...] + degB_ref[...]
        di = jnp.where(d > 0, 1.0 / jnp.sqrt(jnp.maximum(d, 1.0)), 0.0)
        dis_ref[...] = di
        dis2_ref[...] = di * di
        a1_ref[...] = di * e1_ref[...]

    R = 3136
    G = NPAD // R
    return pl.pallas_call(
        body,
        grid=(2, G),
        in_specs=[
            pl.BlockSpec((R, 1), lambda j, i: (i, 0)),
            pl.BlockSpec((R, 1), lambda j, i: (G + i, 0)),
            pl.BlockSpec((R, HD), lambda j, i: (j * G + i, 0)),
        ],
        out_specs=[
            pl.BlockSpec((R, 1), lambda j, i: (i, 0)),
            pl.BlockSpec((R, 1), lambda j, i: (i, 0)),
            pl.BlockSpec((R, HD), lambda j, i: (j * G + i, 0)),
        ],
        out_shape=[
            jax.ShapeDtypeStruct((NPAD, 1), jnp.float32),
            jax.ShapeDtypeStruct((NPAD, 1), jnp.float32),
            jax.ShapeDtypeStruct((2 * NPAD, HD), jnp.float32),
        ],
    )(deg2d, deg2d, E1c)


def _tc_scale1(dis2, y1):
    def body(dis2_ref, y_ref, a_ref):
        a_ref[...] = dis2_ref[...] * y_ref[...]

    R = 3136
    G = NPAD // R
    return pl.pallas_call(
        body,
        grid=(2, G),
        in_specs=[
            pl.BlockSpec((R, 1), lambda j, i: (i, 0)),
            pl.BlockSpec((R, HD), lambda j, i: (j * G + i, 0)),
        ],
        out_specs=pl.BlockSpec((R, HD), lambda j, i: (j * G + i, 0)),
        out_shape=jax.ShapeDtypeStruct((2 * NPAD, HD), jnp.float32),
    )(dis2, y1)


def _tc_fuse(E1p, y1, y2, dis, E2p, W0, b0, W1, b1, attn_W, attn_b, q_W):
    def body(e1_ref, y1lo_ref, y1hi_ref, y2lo_ref, y2hi_ref, dis_ref, e2_ref,
             w0_ref, b0_ref, w1_ref, b1_ref, aw_ref, ab_ref, qw_ref, z_ref):
        di = dis_ref[...]
        y1v = jnp.concatenate([y1lo_ref[...], y1hi_ref[...]], axis=1)
        y2v = jnp.concatenate([y2lo_ref[...], y2hi_ref[...]], axis=1)
        z_p = (e1_ref[...] + di * y1v + di * y2v) / 3.0
        h = jnp.maximum(
            jnp.dot(e2_ref[...], w0_ref[...],
                    preferred_element_type=jnp.float32) + b0_ref[...], 0.0)
        z_n = jnp.maximum(
            jnp.dot(h, w1_ref[...],
                    preferred_element_type=jnp.float32) + b1_ref[...], 0.0)
        aw = aw_ref[...]
        ab = ab_ref[...]
        qw = qw_ref[...]
        w_p = jnp.dot(jnp.tanh(
            jnp.dot(z_p, aw, preferred_element_type=jnp.float32) + ab), qw,
            preferred_element_type=jnp.float32)
        w_n = jnp.dot(jnp.tanh(
            jnp.dot(z_n, aw, preferred_element_type=jnp.float32) + ab), qw,
            preferred_element_type=jnp.float32)
        m = jnp.maximum(w_p, w_n)
        e_p = jnp.exp(w_p - m)
        e_n = jnp.exp(w_n - m)
        alpha = e_p / (e_p + e_n)
        z_ref[...] = alpha * z_p + (1.0 - alpha) * z_n

    R = 3136
    G = NPAD // R
    full = lambda shape: pl.BlockSpec(shape, lambda i: (0, 0))
    return pl.pallas_call(
        body,
        grid=(G,),
        in_specs=[
            pl.BlockSpec((R, D), lambda i: (i, 0)),
            pl.BlockSpec((R, HD), lambda i: (i, 0)),
            pl.BlockSpec((R, HD), lambda i: (G + i, 0)),
            pl.BlockSpec((R, HD), lambda i: (i, 0)),
            pl.BlockSpec((R, HD), lambda i: (G + i, 0)),
            pl.BlockSpec((R, 1), lambda i: (i, 0)),
            pl.BlockSpec((R, D), lambda i: (i, 0)),
            full((D, D)), full((1, D)), full((D, D)), full((1, D)),
            full((D, D)), full((1, D)), full((D, 1)),
        ],
        out_specs=pl.BlockSpec((R, D), lambda i: (i, 0)),
        out_shape=jax.ShapeDtypeStruct((NPAD, D), jnp.float32),
    )(E1p, y1, y1, y2, y2, dis, E2p, W0, b0, W1, b1, attn_W, attn_b, q_W)


def _tc_loss(uv, nmat, w2d):
    R = 256

    def body(u_ref, v_ref, n_ref, w_ref, out_ref):
        i = pl.program_id(0)
        u = u_ref[...]
        v = v_ref[...]
        nb = n_ref[...].reshape(R, K, D)
        wv = w_ref[...]
        pos = jnp.sum(u * v, axis=1, keepdims=True)
        neg = jnp.sum(u[:, None, :] * nb, axis=2)
        coef = -0.5 * jnp.sign(wv) + 1.5
        x = coef * pos - neg
        bpr = jax.nn.log_sigmoid(x)
        reg = jnp.sum(u * u) + jnp.sum(v * v) + jnp.sum(n_ref[...] * n_ref[...])
        part = (-jnp.sum(bpr) + REG * reg).reshape(1, 1)

        @pl.when(i == 0)
        def _():
            out_ref[...] = jnp.zeros((1, 1), jnp.float32)

        out_ref[...] += part

    return pl.pallas_call(
        body,
        grid=(B // R,),
        in_specs=[
            pl.BlockSpec((R, D), lambda i: (i, 0)),
            pl.BlockSpec((R, D), lambda i: (16 + i, 0)),
            pl.BlockSpec((R * K, D), lambda i: (i, 0)),
            pl.BlockSpec((R, 1), lambda i: (i, 0)),
        ],
        out_specs=pl.BlockSpec((1, 1), lambda i: (0, 0)),
        out_shape=jax.ShapeDtypeStruct((1, 1), jnp.float32),
    )(uv, uv, nmat, w2d)


def kernel(E1, E2, W0, b0, W1, b1, attn_W, attn_b, q_W, w, edge_index, u, v, n):
    row = edge_index[0].astype(jnp.int32)
    col = edge_index[1].astype(jnp.int32)
    npad_e = EPAD - E
    pad_row = (jnp.arange(npad_e, dtype=jnp.int32) % 16) + N
    pad_col = jnp.full((npad_e,), -1, jnp.int32)
    rowp = jnp.concatenate([row, pad_row])
    colp = jnp.concatenate([col, pad_col])

    E1p = jnp.pad(E1, ((0, NPAD - N), (0, 0)))
    E2p = jnp.pad(E2, ((0, NPAD - N), (0, 0)))
    E1c = jnp.concatenate([E1p[:, :HD], E1p[:, HD:]], axis=0)

    deg2 = _sc_degree(colp)
    dis, dis2, a1 = _tc_scale0(deg2.reshape(2 * NPAD, 1), E1c)
    y1 = _sc_spass(a1, colp, rowp)
    a2 = _tc_scale1(dis2, y1)
    y2 = _sc_spass(a2, colp, rowp)
    Z = _tc_fuse(E1p, y1, y2, dis, E2p,
                 W0, b0.reshape(1, D), W1, b1.reshape(1, D),
                 attn_W, attn_b.reshape(1, D), q_W)

    idx_all = jnp.concatenate([
        u.astype(jnp.int32), v.astype(jnp.int32),
        n.reshape(-1).astype(jnp.int32)])
    uv, nmat = _sc_gather(Z, idx_all)

    out = _tc_loss(uv, nmat, w2d=w.reshape(B, 1))
    return out[0, 0]


# speedup vs baseline: 17.0789x; 1.0065x over previous
import functools

import jax
import jax.numpy as jnp
from jax import lax
from jax.experimental import pallas as pl
from jax.experimental.pallas import tpu as pltpu
from jax.experimental.pallas import tpu_sc as plsc

N = 50000
D = 64
E = 800000
B = 4096
K = 40
REG = 0.0001

NPAD = 50176
HD = D // 2
ACC = 50688
EPAD = 819200
W = 128
NCHUNK = EPAD // 16 // W
DCHUNK = EPAD // 32 // W

_mesh = lambda: plsc.VectorSubcoreMesh(core_axis_name="c", subcore_axis_name="s")
_SC_PARAMS = pltpu.CompilerParams(use_tc_tiling_on_sc=False)


def _local_idx(colbuf, idxbuf):
    lanes = lax.iota(jnp.int32, 16)
    for k in range(W // 16):
        sl = pl.ds(k * 16, 16)
        cv = colbuf[sl]
        idxbuf[sl] = jnp.where(cv >= 0, cv, NPAD + lanes)


def _wait(src, dst, sem):
    pltpu.make_async_copy(src, dst, sem).wait()


def _sc_degree(colp):
    @functools.partial(
        pl.kernel,
        out_type=jax.ShapeDtypeStruct((2 * NPAD,), jnp.float32),
        mesh=_mesh(),
        compiler_params=_SC_PARAMS,
        scratch_types=[
            pltpu.VMEM((2, W), jnp.int32),
            pltpu.VMEM((2, W), jnp.int32),
            pltpu.VMEM((W,), jnp.float32),
            pltpu.VMEM((3168,), jnp.float32),
            pltpu.VMEM_SHARED((ACC,), jnp.float32),
            pltpu.SemaphoreType.DMA,
            pltpu.SemaphoreType.DMA,
            pltpu.SemaphoreType.DMA,
            pltpu.SemaphoreType.DMA,
        ],
    )
    def k(col_hbm, deg_hbm, colbuf, idxbuf, ones, zbuf, acc,
          lsem0, lsem1, ssem0, ssem1):
        c = lax.axis_index("c")
        s = lax.axis_index("s")

        @pl.loop(0, 3168 // 16)
        def _(k16):
            zbuf[pl.ds(k16 * 16, 16)] = jnp.zeros((16,), jnp.float32)

        for k16 in range(W // 16):
            ones[pl.ds(k16 * 16, 16)] = jnp.ones((16,), jnp.float32)
        pltpu.sync_copy(zbuf, acc.at[pl.ds(s * 3168, 3168)])
        plsc.subcore_barrier()

        def load(b, g, ls):
            e0 = c * (EPAD // 2) + s * (DCHUNK * W) + g * W
            pltpu.async_copy(col_hbm.at[pl.ds(e0, W)], colbuf.at[b], ls)

        load(0, 0, lsem0)
        load(1, 1, lsem1)

        @pl.loop(0, DCHUNK // 2)
        def _(t):
            for b, ls, ss in ((0, lsem0, ssem0), (1, lsem1, ssem1)):
                g = t * 2 + b
                _wait(col_hbm.at[pl.ds(0, W)], colbuf.at[b], ls)

                @pl.when(t > 0)
                def _():
                    _wait(ones, acc.at[idxbuf.at[b]], ss)

                _local_idx(colbuf.at[b], idxbuf.at[b])
                pltpu.async_copy(ones, acc.at[idxbuf.at[b]], ss, add=True)

                @pl.when(t < DCHUNK // 2 - 1)
                def _():
                    load(b, g + 2, ls)

        _wait(ones, acc.at[idxbuf.at[0]], ssem0)
        _wait(ones, acc.at[idxbuf.at[1]], ssem1)
        plsc.subcore_barrier()
        pltpu.sync_copy(acc.at[pl.ds(s * 3136, 3136)], zbuf.at[pl.ds(0, 3136)])
        pltpu.sync_copy(zbuf.at[pl.ds(0, 3136)],
                        deg_hbm.at[pl.ds(c * NPAD + s * 3136, 3136)])

    return k(colp)


def _sc_spass(a2, colp, rowp):
    @functools.partial(
        pl.kernel,
        out_type=jax.ShapeDtypeStruct((2 * NPAD, HD), jnp.float32),
        mesh=_mesh(),
        compiler_params=_SC_PARAMS,
        scratch_types=[
            pltpu.VMEM((4, W), jnp.int32),
            pltpu.VMEM((4, W), jnp.int32),
            pltpu.VMEM((2, W), jnp.int32),
            pltpu.VMEM((2, W, HD), jnp.float32),
            pltpu.VMEM((96, HD), jnp.float32),
            pltpu.VMEM_SHARED((ACC, HD), jnp.float32),
            pltpu.SemaphoreType.DMA,
            pltpu.SemaphoreType.DMA,
            pltpu.SemaphoreType.DMA,
            pltpu.SemaphoreType.DMA,
            pltpu.SemaphoreType.DMA,
            pltpu.SemaphoreType.DMA,
            pltpu.SemaphoreType.DMA,
            pltpu.SemaphoreType.DMA,
        ],
    )
    def k(a_hbm, col_hbm, row_hbm, y_hbm, colbuf, rowbuf, idxbuf, gbuf, zbuf,
          acc, lsem0, lsem1, lsem2, lsem3, gsem0, gsem1, ssem0, ssem1):
        lsem = (lsem0, lsem1, lsem2, lsem3)
        gsem = (gsem0, gsem1)
        ssem = (ssem0, ssem1)
        c = lax.axis_index("c")
        s = lax.axis_index("s")
        roff = c * NPAD

        @pl.loop(0, 96)
        def _(r):
            for k16 in range(HD // 16):
                zbuf[r, pl.ds(k16 * 16, 16)] = jnp.zeros((16,), jnp.float32)

        @pl.loop(0, 3168 // 96)
        def _(j):
            pltpu.sync_copy(zbuf, acc.at[pl.ds(s * 3168 + j * 96, 96)])

        plsc.subcore_barrier()

        def load(slot, g):
            e0 = s * (NCHUNK * W) + g * W
            pltpu.async_copy(col_hbm.at[pl.ds(e0, W)], colbuf.at[slot],
                             lsem[slot])
            pltpu.async_copy(row_hbm.at[pl.ds(e0, W)], rowbuf.at[slot],
                             lsem[slot])

        def wait_load(slot):
            _wait(col_hbm.at[pl.ds(0, W)], colbuf.at[slot], lsem[slot])
            _wait(row_hbm.at[pl.ds(0, W)], rowbuf.at[slot], lsem[slot])

        def wait_scat(b2):
            _wait(gbuf.at[b2], acc.at[idxbuf.at[b2]], ssem[b2])

        def issue_scat(b2):
            pltpu.async_copy(gbuf.at[b2], acc.at[idxbuf.at[b2]], ssem[b2],
                             add=True)

        def wait_gather(slot, b2):
            _wait(a_hbm.at[rowbuf.at[slot]], gbuf.at[b2], gsem[b2])

        def adjust_rows(slot):
            for k16 in range(W // 16):
                sl = pl.ds(k16 * 16, 16)
                rowbuf[slot, sl] = rowbuf[slot, sl] + roff

        load(0, 0)
        load(1, 1)
        load(2, 2)

        @pl.loop(0, NCHUNK // 4)
        def _(t):
            for u in range(4):
                g = t * 4 + u
                b2 = u % 2
                pu = (u + 3) % 4
                wait_load(u)
                if u >= 2:
                    wait_scat(b2)
                else:
                    @pl.when(t > 0)
                    def _():
                        wait_scat(b2)
                _local_idx(colbuf.at[u], idxbuf.at[b2])
                adjust_rows(u)
                pltpu.async_copy(a_hbm.at[rowbuf.at[u]], gbuf.at[b2],
                                 gsem[b2])
                if u >= 1:
                    wait_gather(pu, 1 - b2)
                    issue_scat(1 - b2)
                    @pl.when(g + 3 < NCHUNK)
                    def _():
                        load(pu, g + 3)
                else:
                    @pl.when(t > 0)
                    def _():
                        wait_gather(3, 1)
                        issue_scat(1)
                    load(3, g + 3)

        wait_gather(3, 1)
        issue_scat(1)
        wait_scat(0)
        wait_scat(1)
        plsc.subcore_barrier()
        pltpu.sync_copy(
            acc.at[pl.ds(s * 3136, 3136)],
            y_hbm.at[pl.ds(c * NPAD + s * 3136, 3136)],
        )

    return k(a2, colp, rowp)


def _sc_gather(Z, idx_all):
    NUV = 2 * B
    NNEG = B * K
    TOT = NUV + NNEG
    WIN_PER_TILE = TOT // (32 * W)
    UV_WINS = NUV // W

    @functools.partial(
        pl.kernel,
        out_type=(
            jax.ShapeDtypeStruct((NUV, D), jnp.float32),
            jax.ShapeDtypeStruct((NNEG, D), jnp.float32),
        ),
        mesh=_mesh(),
        compiler_params=_SC_PARAMS,
        scratch_types=[
            pltpu.VMEM((2, W), jnp.int32),
            pltpu.VMEM((2, W, D), jnp.float32),
            pltpu.SemaphoreType.DMA,
            pltpu.SemaphoreType.DMA,
            pltpu.SemaphoreType.DMA,
            pltpu.SemaphoreType.DMA,
            pltpu.SemaphoreType.DMA,
            pltpu.SemaphoreType.DMA,
        ],
    )
    def k(z_hbm, idx_hbm, uv_hbm, n_hbm, idxbuf, gbuf,
          lsem0, lsem1, gsem0, gsem1, osem0, osem1):
        c = lax.axis_index("c")
        s = lax.axis_index("s")
        wid = s * 2 + c

        def load(b, j, ls):
            wabs = wid * WIN_PER_TILE + j
            pltpu.async_copy(idx_hbm.at[pl.ds(wabs * W, W)], idxbuf.at[b], ls)

        load(0, 0, lsem0)
        load(1, 1, lsem1)

        @pl.loop(0, WIN_PER_TILE // 2)
        def _(t):
            for b, ls, gs, os in ((0, lsem0, gsem0, osem0),
                                  (1, lsem1, gsem1, osem1)):
                j = t * 2 + b
                wabs = wid * WIN_PER_TILE + j
                _wait(idx_hbm.at[pl.ds(0, W)], idxbuf.at[b], ls)

                @pl.when(t > 0)
                def _():
                    _wait(gbuf.at[b], uv_hbm.at[pl.ds(0, W)], os)

                pltpu.async_copy(z_hbm.at[idxbuf.at[b]], gbuf.at[b], gs)
                _wait(z_hbm.at[idxbuf.at[b]], gbuf.at[b], gs)

                @pl.when(wabs < UV_WINS)
                def _():
                    pltpu.async_copy(gbuf.at[b], uv_hbm.at[pl.ds(wabs * W, W)],
                                     os)

                @pl.when(wabs >= UV_WINS)
                def _():
                    pltpu.async_copy(
                        gbuf.at[b], n_hbm.at[pl.ds(wabs * W - NUV, W)], os)

                @pl.when(t < WIN_PER_TILE // 2 - 1)
                def _():
                    load(b, j + 2, ls)

        _wait(gbuf.at[0], uv_hbm.at[pl.ds(0, W)], osem0)
        _wait(gbuf.at[1], uv_hbm.at[pl.ds(0, W)], osem1)

    return k(Z, idx_all)


def _tc_scale0(deg2d, E1c):
    def body(degA_ref, degB_ref, e1lo_ref, e1hi_ref, dis_ref, dis2_ref,
             a1_ref):
        d = degA_ref[...] + degB_ref[...]
        di = jnp.where(d > 0, 1.0 / jnp.sqrt(jnp.maximum(d, 1.0)), 0.0)
        dis_ref[...] = di
        dis2_ref[...] = di * di
        a1_ref[0] = di * e1lo_ref[...]
        a1_ref[1] = di * e1hi_ref[...]

    R = 3136
    G = NPAD // R
    return pl.pallas_call(
        body,
        grid=(G,),
        in_specs=[
            pl.BlockSpec((R, 1), lambda i: (i, 0)),
            pl.BlockSpec((R, 1), lambda i: (G + i, 0)),
            pl.BlockSpec((R, HD), lambda i: (i, 0)),
            pl.BlockSpec((R, HD), lambda i: (G + i, 0)),
        ],
        out_specs=[
            pl.BlockSpec((R, 1), lambda i: (i, 0)),
            pl.BlockSpec((R, 1), lambda i: (i, 0)),
            pl.BlockSpec((2, R, HD), lambda i: (0, i, 0)),
        ],
        out_shape=[
            jax.ShapeDtypeStruct((NPAD, 1), jnp.float32),
            jax.ShapeDtypeStruct((NPAD, 1), jnp.float32),
            jax.ShapeDtypeStruct((2, NPAD, HD), jnp.float32),
        ],
    )(deg2d, deg2d, E1c, E1c)


def _tc_scale1(dis2, y1):
    def body(dis2_ref, y_ref, a_ref):
        a_ref[...] = dis2_ref[...] * y_ref[...]

    R = 3136
    G = NPAD // R
    return pl.pallas_call(
        body,
        grid=(2, G),
        in_specs=[
            pl.BlockSpec((R, 1), lambda j, i: (i, 0)),
            pl.BlockSpec((R, HD), lambda j, i: (j * G + i, 0)),
        ],
        out_specs=pl.BlockSpec((R, HD), lambda j, i: (j * G + i, 0)),
        out_shape=jax.ShapeDtypeStruct((2 * NPAD, HD), jnp.float32),
    )(dis2, y1)


def _tc_mlp(E2p, W0, b0, W1, b1, attn_W, attn_b, q_W):
    def body(e2_ref, w0_ref, b0_ref, w1_ref, b1_ref, aw_ref, ab_ref, qw_ref,
             zn_ref, wn_ref):
        h = jnp.maximum(
            jnp.dot(e2_ref[...], w0_ref[...],
                    preferred_element_type=jnp.float32) + b0_ref[...], 0.0)
        z_n = jnp.maximum(
            jnp.dot(h, w1_ref[...],
                    preferred_element_type=jnp.float32) + b1_ref[...], 0.0)
        zn_ref[...] = z_n
        wn_ref[...] = jnp.dot(jnp.tanh(
            jnp.dot(z_n, aw_ref[...], preferred_element_type=jnp.float32)
            + ab_ref[...]), qw_ref[...], preferred_element_type=jnp.float32)

    R = 3136
    G = NPAD // R
    full = lambda shape: pl.BlockSpec(shape, lambda i: (0, 0))
    return pl.pallas_call(
        body,
        grid=(G,),
        in_specs=[
            pl.BlockSpec((R, D), lambda i: (i, 0)),
            full((D, D)), full((1, D)), full((D, D)), full((1, D)),
            full((D, D)), full((1, D)), full((D, 1)),
        ],
        out_specs=[
            pl.BlockSpec((R, D), lambda i: (i, 0)),
            pl.BlockSpec((R, 1), lambda i: (i, 0)),
        ],
        out_shape=[
            jax.ShapeDtypeStruct((NPAD, D), jnp.float32),
            jax.ShapeDtypeStruct((NPAD, 1), jnp.float32),
        ],
    )(E2p, W0, b0, W1, b1, attn_W, attn_b, q_W)


def _tc_fuse(E1c, y1, y2, dis, z_n, w_n, attn_W, attn_b, q_W):
    def body(e1lo_ref, e1hi_ref, y1lo_ref, y1hi_ref, y2lo_ref, y2hi_ref,
             dis_ref, zn_ref, wn_ref, aw_ref, ab_ref, qw_ref, z_ref):
        di = dis_ref[...]
        e1v = jnp.concatenate([e1lo_ref[...], e1hi_ref[...]], axis=1)
        y1v = jnp.concatenate([y1lo_ref[...], y1hi_ref[...]], axis=1)
        y2v = jnp.concatenate([y2lo_ref[...], y2hi_ref[...]], axis=1)
        z_p = (e1v + di * y1v + di * y2v) / 3.0
        w_p = jnp.dot(jnp.tanh(
            jnp.dot(z_p, aw_ref[...], preferred_element_type=jnp.float32)
            + ab_ref[...]), qw_ref[...], preferred_element_type=jnp.float32)
        w_n = wn_ref[...]
        m = jnp.maximum(w_p, w_n)
        e_p = jnp.exp(w_p - m)
        e_n = jnp.exp(w_n - m)
        alpha = e_p / (e_p + e_n)
        z_ref[...] = alpha * z_p + (1.0 - alpha) * zn_ref[...]

    R = 3136
    G = NPAD // R
    full = lambda shape: pl.BlockSpec(shape, lambda i: (0, 0))
    return pl.pallas_call(
        body,
        grid=(G,),
        in_specs=[
            pl.BlockSpec((R, HD), lambda i: (i, 0)),
            pl.BlockSpec((R, HD), lambda i: (G + i, 0)),
            pl.BlockSpec((R, HD), lambda i: (i, 0)),
            pl.BlockSpec((R, HD), lambda i: (G + i, 0)),
            pl.BlockSpec((R, HD), lambda i: (i, 0)),
            pl.BlockSpec((R, HD), lambda i: (G + i, 0)),
            pl.BlockSpec((R, 1), lambda i: (i, 0)),
            pl.BlockSpec((R, D), lambda i: (i, 0)),
            pl.BlockSpec((R, 1), lambda i: (i, 0)),
            full((D, D)), full((1, D)), full((D, 1)),
        ],
        out_specs=pl.BlockSpec((R, D), lambda i: (i, 0)),
        out_shape=jax.ShapeDtypeStruct((NPAD, D), jnp.float32),
    )(E1c, E1c, y1, y1, y2, y2, dis, z_n, w_n, attn_W, attn_b, q_W)


def _tc_loss(uv, nmat, w2d):
    R = 256

    def body(u_ref, v_ref, n_ref, w_ref, out_ref):
        i = pl.program_id(0)
        u = u_ref[...]
        v = v_ref[...]
        nb = n_ref[...].reshape(R, K, D)
        wv = w_ref[...]
        pos = jnp.sum(u * v, axis=1, keepdims=True)
        neg = jnp.sum(u[:, None, :] * nb, axis=2)
        coef = -0.5 * jnp.sign(wv) + 1.5
        x = coef * pos - neg
        bpr = jax.nn.log_sigmoid(x)
        reg = jnp.sum(u * u) + jnp.sum(v * v) + jnp.sum(n_ref[...] * n_ref[...])
        part = (-jnp.sum(bpr) + REG * reg).reshape(1, 1)

        @pl.when(i == 0)
        def _():
            out_ref[...] = jnp.zeros((1, 1), jnp.float32)

        out_ref[...] += part

    return pl.pallas_call(
        body,
        grid=(B // R,),
        in_specs=[
            pl.BlockSpec((R, D), lambda i: (i, 0)),
            pl.BlockSpec((R, D), lambda i: (16 + i, 0)),
            pl.BlockSpec((R * K, D), lambda i: (i, 0)),
            pl.BlockSpec((R, 1), lambda i: (i, 0)),
        ],
        out_specs=pl.BlockSpec((1, 1), lambda i: (0, 0)),
        out_shape=jax.ShapeDtypeStruct((1, 1), jnp.float32),
    )(uv, uv, nmat, w2d)


def kernel(E1, E2, W0, b0, W1, b1, attn_W, attn_b, q_W, w, edge_index, u, v, n):
    row = edge_index[0].astype(jnp.int32)
    col = edge_index[1].astype(jnp.int32)
    npad_e = EPAD - E
    pad_row = (jnp.arange(npad_e, dtype=jnp.int32) % 16) + N
    pad_col = jnp.full((npad_e,), -1, jnp.int32)
    rowp = jnp.concatenate([row, pad_row])
    colp = jnp.concatenate([col, pad_col])

    E2p = jnp.pad(E2, ((0, NPAD - N), (0, 0)))
    E1c = jnp.concatenate(
        [jnp.pad(E1[:, :HD], ((0, NPAD - N), (0, 0))),
         jnp.pad(E1[:, HD:], ((0, NPAD - N), (0, 0)))], axis=0)

    z_n, w_n = _tc_mlp(E2p, W0, b0.reshape(1, D), W1, b1.reshape(1, D),
                       attn_W, attn_b.reshape(1, D), q_W)
    deg2 = _sc_degree(colp)
    dis, dis2, a1 = _tc_scale0(deg2.reshape(2 * NPAD, 1), E1c)
    y1 = _sc_spass(a1.reshape(2 * NPAD, HD), colp, rowp)
    a2 = _tc_scale1(dis2, y1)
    y2 = _sc_spass(a2, colp, rowp)
    Z = _tc_fuse(E1c, y1, y2, dis, z_n, w_n,
                 attn_W, attn_b.reshape(1, D), q_W)

    idx_all = jnp.concatenate([
        u.astype(jnp.int32), v.astype(jnp.int32),
        n.reshape(-1).astype(jnp.int32)])
    uv, nmat = _sc_gather(Z, idx_all)

    out = _tc_loss(uv, nmat, w2d=w.reshape(B, 1))
    return out[0, 0]


# speedup vs baseline: 17.9562x; 1.0514x over previous
import functools

import jax
import jax.numpy as jnp
from jax import lax
from jax.experimental import pallas as pl
from jax.experimental.pallas import tpu as pltpu
from jax.experimental.pallas import tpu_sc as plsc

N = 50000
D = 64
E = 800000
B = 4096
K = 40
REG = 0.0001

NPAD = 50176
HD = D // 2
ACC = 50688
EPAD = 819200
W = 128
NCHUNK = EPAD // 16 // W
DCHUNK = EPAD // 32 // W

_mesh = lambda: plsc.VectorSubcoreMesh(core_axis_name="c", subcore_axis_name="s")
_SC_PARAMS = pltpu.CompilerParams(use_tc_tiling_on_sc=False)


def _local_idx(colbuf, idxbuf):
    lanes = lax.iota(jnp.int32, 16)
    for k in range(W // 16):
        sl = pl.ds(k * 16, 16)
        cv = colbuf[sl]
        idxbuf[sl] = jnp.where(cv >= 0, cv, NPAD + lanes)


def _wait(src, dst, sem):
    pltpu.make_async_copy(src, dst, sem).wait()


def _sc_degree(colp):
    @functools.partial(
        pl.kernel,
        out_type=jax.ShapeDtypeStruct((2 * NPAD,), jnp.float32),
        mesh=_mesh(),
        compiler_params=_SC_PARAMS,
        scratch_types=[
            pltpu.VMEM((2, W), jnp.int32),
            pltpu.VMEM((2, W), jnp.int32),
            pltpu.VMEM((W,), jnp.float32),
            pltpu.VMEM((3168,), jnp.float32),
            pltpu.VMEM_SHARED((ACC,), jnp.float32),
            pltpu.SemaphoreType.DMA,
            pltpu.SemaphoreType.DMA,
            pltpu.SemaphoreType.DMA,
            pltpu.SemaphoreType.DMA,
        ],
    )
    def k(col_hbm, deg_hbm, colbuf, idxbuf, ones, zbuf, acc,
          lsem0, lsem1, ssem0, ssem1):
        c = lax.axis_index("c")
        s = lax.axis_index("s")

        @pl.loop(0, 3168 // 16)
        def _(k16):
            zbuf[pl.ds(k16 * 16, 16)] = jnp.zeros((16,), jnp.float32)

        for k16 in range(W // 16):
            ones[pl.ds(k16 * 16, 16)] = jnp.ones((16,), jnp.float32)
        pltpu.sync_copy(zbuf, acc.at[pl.ds(s * 3168, 3168)])
        plsc.subcore_barrier()

        def load(b, g, ls):
            e0 = c * (EPAD // 2) + s * (DCHUNK * W) + g * W
            pltpu.async_copy(col_hbm.at[pl.ds(e0, W)], colbuf.at[b], ls)

        load(0, 0, lsem0)
        load(1, 1, lsem1)

        @pl.loop(0, DCHUNK // 2)
        def _(t):
            for b, ls, ss in ((0, lsem0, ssem0), (1, lsem1, ssem1)):
                g = t * 2 + b
                _wait(col_hbm.at[pl.ds(0, W)], colbuf.at[b], ls)

                @pl.when(t > 0)
                def _():
                    _wait(ones, acc.at[idxbuf.at[b]], ss)

                _local_idx(colbuf.at[b], idxbuf.at[b])
                pltpu.async_copy(ones, acc.at[idxbuf.at[b]], ss, add=True)

                @pl.when(t < DCHUNK // 2 - 1)
                def _():
                    load(b, g + 2, ls)

        _wait(ones, acc.at[idxbuf.at[0]], ssem0)
        _wait(ones, acc.at[idxbuf.at[1]], ssem1)
        plsc.subcore_barrier()
        pltpu.sync_copy(acc.at[pl.ds(s * 3136, 3136)], zbuf.at[pl.ds(0, 3136)])
        pltpu.sync_copy(zbuf.at[pl.ds(0, 3136)],
                        deg_hbm.at[pl.ds(c * NPAD + s * 3136, 3136)])

    return k(colp)


def _sc_spass(a2, colp, rowp, dis2=None):
    out_type = jax.ShapeDtypeStruct((2 * NPAD, HD), jnp.float32)
    if dis2 is not None:
        out_type = (out_type, jax.ShapeDtypeStruct((2 * NPAD, HD), jnp.float32))

    @functools.partial(
        pl.kernel,
        out_type=out_type,
        mesh=_mesh(),
        compiler_params=_SC_PARAMS,
        scratch_types=[
            pltpu.VMEM((4, W), jnp.int32),
            pltpu.VMEM((4, W), jnp.int32),
            pltpu.VMEM((2, W), jnp.int32),
            pltpu.VMEM((2, W, HD), jnp.float32),
            pltpu.VMEM((96, HD), jnp.float32),
            pltpu.VMEM((3136,), jnp.float32),
            pltpu.VMEM_SHARED((ACC, HD), jnp.float32),
            pltpu.SemaphoreType.DMA,
            pltpu.SemaphoreType.DMA,
            pltpu.SemaphoreType.DMA,
            pltpu.SemaphoreType.DMA,
            pltpu.SemaphoreType.DMA,
            pltpu.SemaphoreType.DMA,
            pltpu.SemaphoreType.DMA,
            pltpu.SemaphoreType.DMA,
        ],
    )
    def k(a_hbm, col_hbm, row_hbm, *rest):
        if dis2 is None:
            (y_hbm, colbuf, rowbuf, idxbuf, gbuf, zbuf, dbuf, acc,
             lsem0, lsem1, lsem2, lsem3, gsem0, gsem1, ssem0, ssem1) = rest
            d2_hbm = a2_hbm = None
        else:
            (d2_hbm, y_hbm, a2_hbm, colbuf, rowbuf, idxbuf, gbuf, zbuf, dbuf,
             acc, lsem0, lsem1, lsem2, lsem3, gsem0, gsem1, ssem0, ssem1) = rest
        lsem = (lsem0, lsem1, lsem2, lsem3)
        gsem = (gsem0, gsem1)
        ssem = (ssem0, ssem1)
        c = lax.axis_index("c")
        s = lax.axis_index("s")
        roff = c * NPAD

        @pl.loop(0, 96)
        def _(r):
            for k16 in range(HD // 16):
                zbuf[r, pl.ds(k16 * 16, 16)] = jnp.zeros((16,), jnp.float32)

        @pl.loop(0, 3168 // 96)
        def _(j):
            pltpu.sync_copy(zbuf, acc.at[pl.ds(s * 3168 + j * 96, 96)])

        plsc.subcore_barrier()

        def load(slot, g):
            e0 = s * (NCHUNK * W) + g * W
            pltpu.async_copy(col_hbm.at[pl.ds(e0, W)], colbuf.at[slot],
                             lsem[slot])
            pltpu.async_copy(row_hbm.at[pl.ds(e0, W)], rowbuf.at[slot],
                             lsem[slot])

        def wait_load(slot):
            _wait(col_hbm.at[pl.ds(0, W)], colbuf.at[slot], lsem[slot])
            _wait(row_hbm.at[pl.ds(0, W)], rowbuf.at[slot], lsem[slot])

        def wait_scat(b2):
            _wait(gbuf.at[b2], acc.at[idxbuf.at[b2]], ssem[b2])

        def issue_scat(b2):
            pltpu.async_copy(gbuf.at[b2], acc.at[idxbuf.at[b2]], ssem[b2],
                             add=True)

        def wait_gather(slot, b2):
            _wait(a_hbm.at[rowbuf.at[slot]], gbuf.at[b2], gsem[b2])

        def adjust_rows(slot):
            for k16 in range(W // 16):
                sl = pl.ds(k16 * 16, 16)
                rowbuf[slot, sl] = rowbuf[slot, sl] + roff

        load(0, 0)
        load(1, 1)
        load(2, 2)

        @pl.loop(0, NCHUNK // 4)
        def _(t):
            for u in range(4):
                g = t * 4 + u
                b2 = u % 2
                pu = (u + 3) % 4
                wait_load(u)
                if u >= 2:
                    wait_scat(b2)
                else:
                    @pl.when(t > 0)
                    def _():
                        wait_scat(b2)
                _local_idx(colbuf.at[u], idxbuf.at[b2])
                adjust_rows(u)
                pltpu.async_copy(a_hbm.at[rowbuf.at[u]], gbuf.at[b2],
                                 gsem[b2])
                if u >= 1:
                    wait_gather(pu, 1 - b2)
                    issue_scat(1 - b2)
                    @pl.when(g + 3 < NCHUNK)
                    def _():
                        load(pu, g + 3)
                else:
                    @pl.when(t > 0)
                    def _():
                        wait_gather(3, 1)
                        issue_scat(1)
                    load(3, g + 3)

        wait_gather(3, 1)
        issue_scat(1)
        wait_scat(0)
        wait_scat(1)
        plsc.subcore_barrier()
        pltpu.sync_copy(
            acc.at[pl.ds(s * 3136, 3136)],
            y_hbm.at[pl.ds(c * NPAD + s * 3136, 3136)],
        )
        if dis2 is not None:
            pltpu.sync_copy(d2_hbm.at[pl.ds(s * 3136, 3136)], dbuf)

            @pl.loop(0, 3136 // 96)
            def _(j):
                pltpu.sync_copy(acc.at[pl.ds(s * 3136 + j * 96, 96)], zbuf)

                @pl.loop(0, 96)
                def _(r):
                    dv = dbuf[pl.ds(j * 96 + r, 1)][0]
                    for k16 in range(HD // 16):
                        sl = pl.ds(k16 * 16, 16)
                        zbuf[r, sl] = zbuf[r, sl] * dv

                pltpu.sync_copy(
                    zbuf,
                    a2_hbm.at[pl.ds(c * NPAD + s * 3136 + j * 96, 96)])

    if dis2 is None:
        return k(a2, colp, rowp)
    return k(a2, colp, rowp, dis2)


def _sc_gather(Z, idx_all):
    NUV = 2 * B
    NNEG = B * K
    TOT = NUV + NNEG
    WIN_PER_TILE = TOT // (32 * W)
    UV_WINS = NUV // W

    @functools.partial(
        pl.kernel,
        out_type=(
            jax.ShapeDtypeStruct((NUV, D), jnp.float32),
            jax.ShapeDtypeStruct((NNEG, D), jnp.float32),
        ),
        mesh=_mesh(),
        compiler_params=_SC_PARAMS,
        scratch_types=[
            pltpu.VMEM((2, W), jnp.int32),
            pltpu.VMEM((2, W, D), jnp.float32),
            pltpu.SemaphoreType.DMA,
            pltpu.SemaphoreType.DMA,
            pltpu.SemaphoreType.DMA,
            pltpu.SemaphoreType.DMA,
            pltpu.SemaphoreType.DMA,
            pltpu.SemaphoreType.DMA,
        ],
    )
    def k(z_hbm, idx_hbm, uv_hbm, n_hbm, idxbuf, gbuf,
          lsem0, lsem1, gsem0, gsem1, osem0, osem1):
        c = lax.axis_index("c")
        s = lax.axis_index("s")
        wid = s * 2 + c

        def load(b, j, ls):
            wabs = wid * WIN_PER_TILE + j
            pltpu.async_copy(idx_hbm.at[pl.ds(wabs * W, W)], idxbuf.at[b], ls)

        load(0, 0, lsem0)
        load(1, 1, lsem1)

        @pl.loop(0, WIN_PER_TILE // 2)
        def _(t):
            for b, ls, gs, os in ((0, lsem0, gsem0, osem0),
                                  (1, lsem1, gsem1, osem1)):
                j = t * 2 + b
                wabs = wid * WIN_PER_TILE + j
                _wait(idx_hbm.at[pl.ds(0, W)], idxbuf.at[b], ls)

                @pl.when(t > 0)
                def _():
                    _wait(gbuf.at[b], uv_hbm.at[pl.ds(0, W)], os)

                pltpu.async_copy(z_hbm.at[idxbuf.at[b]], gbuf.at[b], gs)
                _wait(z_hbm.at[idxbuf.at[b]], gbuf.at[b], gs)

                @pl.when(wabs < UV_WINS)
                def _():
                    pltpu.async_copy(gbuf.at[b], uv_hbm.at[pl.ds(wabs * W, W)],
                                     os)

                @pl.when(wabs >= UV_WINS)
                def _():
                    pltpu.async_copy(
                        gbuf.at[b], n_hbm.at[pl.ds(wabs * W - NUV, W)], os)

                @pl.when(t < WIN_PER_TILE // 2 - 1)
                def _():
                    load(b, j + 2, ls)

        _wait(gbuf.at[0], uv_hbm.at[pl.ds(0, W)], osem0)
        _wait(gbuf.at[1], uv_hbm.at[pl.ds(0, W)], osem1)

    return k(Z, idx_all)


def _tc_scale0(deg2d, E1c):
    def body(degA_ref, degB_ref, e1lo_ref, e1hi_ref, dis_ref, dis2_ref,
             a1_ref):
        d = degA_ref[...] + degB_ref[...]
        di = jnp.where(d > 0, 1.0 / jnp.sqrt(jnp.maximum(d, 1.0)), 0.0)
        dis_ref[...] = di
        dis2_ref[...] = di * di
        a1_ref[0] = di * e1lo_ref[...]
        a1_ref[1] = di * e1hi_ref[...]

    R = 3136
    G = NPAD // R
    return pl.pallas_call(
        body,
        grid=(G,),
        in_specs=[
            pl.BlockSpec((R, 1), lambda i: (i, 0)),
            pl.BlockSpec((R, 1), lambda i: (G + i, 0)),
            pl.BlockSpec((R, HD), lambda i: (i, 0)),
            pl.BlockSpec((R, HD), lambda i: (G + i, 0)),
        ],
        out_specs=[
            pl.BlockSpec((R, 1), lambda i: (i, 0)),
            pl.BlockSpec((R, 1), lambda i: (i, 0)),
            pl.BlockSpec((2, R, HD), lambda i: (0, i, 0)),
        ],
        out_shape=[
            jax.ShapeDtypeStruct((NPAD, 1), jnp.float32),
            jax.ShapeDtypeStruct((NPAD, 1), jnp.float32),
            jax.ShapeDtypeStruct((2, NPAD, HD), jnp.float32),
        ],
    )(deg2d, deg2d, E1c, E1c)


def _tc_mlp(E2p, W0, b0, W1, b1, attn_W, attn_b, q_W):
    def body(e2_ref, w0_ref, b0_ref, w1_ref, b1_ref, aw_ref, ab_ref, qw_ref,
             zn_ref, wn_ref):
        h = jnp.maximum(
            jnp.dot(e2_ref[...], w0_ref[...],
                    preferred_element_type=jnp.float32) + b0_ref[...], 0.0)
        z_n = jnp.maximum(
            jnp.dot(h, w1_ref[...],
                    preferred_element_type=jnp.float32) + b1_ref[...], 0.0)
        zn_ref[...] = z_n
        wn_ref[...] = jnp.dot(jnp.tanh(
            jnp.dot(z_n, aw_ref[...], preferred_element_type=jnp.float32)
            + ab_ref[...]), qw_ref[...], preferred_element_type=jnp.float32)

    R = 3136
    G = NPAD // R
    full = lambda shape: pl.BlockSpec(shape, lambda i: (0, 0))
    return pl.pallas_call(
        body,
        grid=(G,),
        in_specs=[
            pl.BlockSpec((R, D), lambda i: (i, 0)),
            full((D, D)), full((1, D)), full((D, D)), full((1, D)),
            full((D, D)), full((1, D)), full((D, 1)),
        ],
        out_specs=[
            pl.BlockSpec((R, D), lambda i: (i, 0)),
            pl.BlockSpec((R, 1), lambda i: (i, 0)),
        ],
        out_shape=[
            jax.ShapeDtypeStruct((NPAD, D), jnp.float32),
            jax.ShapeDtypeStruct((NPAD, 1), jnp.float32),
        ],
    )(E2p, W0, b0, W1, b1, attn_W, attn_b, q_W)


def _tc_fuse(E1c, y1, y2, dis, z_n, w_n, attn_W, attn_b, q_W):
    def body(e1lo_ref, e1hi_ref, y1lo_ref, y1hi_ref, y2lo_ref, y2hi_ref,
             dis_ref, zn_ref, wn_ref, aw_ref, ab_ref, qw_ref, z_ref):
        di = dis_ref[...]
        e1v = jnp.concatenate([e1lo_ref[...], e1hi_ref[...]], axis=1)
        y1v = jnp.concatenate([y1lo_ref[...], y1hi_ref[...]], axis=1)
        y2v = jnp.concatenate([y2lo_ref[...], y2hi_ref[...]], axis=1)
        z_p = (e1v + di * y1v + di * y2v) / 3.0
        w_p = jnp.dot(jnp.tanh(
            jnp.dot(z_p, aw_ref[...], preferred_element_type=jnp.float32)
            + ab_ref[...]), qw_ref[...], preferred_element_type=jnp.float32)
        w_n = wn_ref[...]
        m = jnp.maximum(w_p, w_n)
        e_p = jnp.exp(w_p - m)
        e_n = jnp.exp(w_n - m)
        alpha = e_p / (e_p + e_n)
        z_ref[...] = alpha * z_p + (1.0 - alpha) * zn_ref[...]

    R = 3136
    G = NPAD // R
    full = lambda shape: pl.BlockSpec(shape, lambda i: (0, 0))
    return pl.pallas_call(
        body,
        grid=(G,),
        in_specs=[
            pl.BlockSpec((R, HD), lambda i: (i, 0)),
            pl.BlockSpec((R, HD), lambda i: (G + i, 0)),
            pl.BlockSpec((R, HD), lambda i: (i, 0)),
            pl.BlockSpec((R, HD), lambda i: (G + i, 0)),
            pl.BlockSpec((R, HD), lambda i: (i, 0)),
            pl.BlockSpec((R, HD), lambda i: (G + i, 0)),
            pl.BlockSpec((R, 1), lambda i: (i, 0)),
            pl.BlockSpec((R, D), lambda i: (i, 0)),
            pl.BlockSpec((R, 1), lambda i: (i, 0)),
            full((D, D)), full((1, D)), full((D, 1)),
        ],
        out_specs=pl.BlockSpec((R, D), lambda i: (i, 0)),
        out_shape=jax.ShapeDtypeStruct((NPAD, D), jnp.float32),
    )(E1c, E1c, y1, y1, y2, y2, dis, z_n, w_n, attn_W, attn_b, q_W)


def _tc_loss(uv, nmat, w2d):
    R = 256

    def body(u_ref, v_ref, n_ref, w_ref, out_ref):
        i = pl.program_id(0)
        u = u_ref[...]
        v = v_ref[...]
        nb = n_ref[...].reshape(R, K, D)
        wv = w_ref[...]
        pos = jnp.sum(u * v, axis=1, keepdims=True)
        neg = jnp.sum(u[:, None, :] * nb, axis=2)
        coef = -0.5 * jnp.sign(wv) + 1.5
        x = coef * pos - neg
        bpr = jax.nn.log_sigmoid(x)
        reg = jnp.sum(u * u) + jnp.sum(v * v) + jnp.sum(n_ref[...] * n_ref[...])
        part = (-jnp.sum(bpr) + REG * reg).reshape(1, 1)

        @pl.when(i == 0)
        def _():
            out_ref[...] = jnp.zeros((1, 1), jnp.float32)

        out_ref[...] += part

    return pl.pallas_call(
        body,
        grid=(B // R,),
        in_specs=[
            pl.BlockSpec((R, D), lambda i: (i, 0)),
            pl.BlockSpec((R, D), lambda i: (16 + i, 0)),
            pl.BlockSpec((R * K, D), lambda i: (i, 0)),
            pl.BlockSpec((R, 1), lambda i: (i, 0)),
        ],
        out_specs=pl.BlockSpec((1, 1), lambda i: (0, 0)),
        out_shape=jax.ShapeDtypeStruct((1, 1), jnp.float32),
    )(uv, uv, nmat, w2d)


def kernel(E1, E2, W0, b0, W1, b1, attn_W, attn_b, q_W, w, edge_index, u, v, n):
    row = edge_index[0].astype(jnp.int32)
    col = edge_index[1].astype(jnp.int32)
    npad_e = EPAD - E
    pad_row = (jnp.arange(npad_e, dtype=jnp.int32) % 16) + N
    pad_col = jnp.full((npad_e,), -1, jnp.int32)
    rowp = jnp.concatenate([row, pad_row])
    colp = jnp.concatenate([col, pad_col])

    E2p = jnp.pad(E2, ((0, NPAD - N), (0, 0)))
    E1c = jnp.concatenate(
        [jnp.pad(E1[:, :HD], ((0, NPAD - N), (0, 0))),
         jnp.pad(E1[:, HD:], ((0, NPAD - N), (0, 0)))], axis=0)

    z_n, w_n = _tc_mlp(E2p, W0, b0.reshape(1, D), W1, b1.reshape(1, D),
                       attn_W, attn_b.reshape(1, D), q_W)
    deg2 = _sc_degree(colp)
    dis, dis2, a1 = _tc_scale0(deg2.reshape(2 * NPAD, 1), E1c)
    y1, a2 = _sc_spass(a1.reshape(2 * NPAD, HD), colp, rowp,
                       dis2=dis2.reshape(NPAD))
    y2 = _sc_spass(a2, colp, rowp)
    Z = _tc_fuse(E1c, y1, y2, dis, z_n, w_n,
                 attn_W, attn_b.reshape(1, D), q_W)

    idx_all = jnp.concatenate([
        u.astype(jnp.int32), v.astype(jnp.int32),
        n.reshape(-1).astype(jnp.int32)])
    uv, nmat = _sc_gather(Z, idx_all)

    out = _tc_loss(uv, nmat, w2d=w.reshape(B, 1))
    return out[0, 0]
